# Initial kernel scaffold; baseline (speedup 1.0000x reference)
import jax
import jax.numpy as jnp
from jax.experimental import pallas as pl


def kernel(b_z, edge_index, b_edge_weight, b_size, W1s, W1n, b1, W2s, W2n, b2):
    raise NotImplementedError("write your pallas kernel here")


# speedup vs baseline: 6.4694x; 6.4694x over previous
import functools

import jax
import jax.numpy as jnp
from jax import lax
from jax.experimental import pallas as pl
from jax.experimental.pallas import tpu as pltpu
from jax.experimental.pallas import tpu_sc as plsc

N = 10000
E = 320000
D = 128
H = 64

NC = 2
NS = 16
NW = NC * NS
EPW = E // NW
C = 400
NSUB = 5
SUB = C // NSUB
NCHUNK = EPW // C
ROWS_PT = 640
NPAD = NS * ROWS_PT


def _zero_f32(ref, nwords):
    z = jnp.zeros((16,), jnp.float32)

    def body(i, c):
        ref[pl.ds(i * 16, 16)] = z
        return c

    lax.fori_loop(0, nwords // 16, body, 0)


def _tc_pre_body(bz_ref, w1n_ref, w1s_ref, b1_ref, y1n_ref, a1s_ref):
    x = bz_ref[...]
    y1n_ref[...] = jnp.dot(x, w1n_ref[...], preferred_element_type=jnp.float32)
    a1s_ref[...] = (
        jnp.dot(x, w1s_ref[...], preferred_element_type=jnp.float32)
        + b1_ref[...]
    )


def _tc_pre(b_z, W1n, W1s, b1row):
    R = 1000
    return pl.pallas_call(
        _tc_pre_body,
        grid=(N // R,),
        in_specs=[
            pl.BlockSpec((R, D), lambda j: (j, 0)),
            pl.BlockSpec((D, H), lambda j: (0, 0)),
            pl.BlockSpec((D, H), lambda j: (0, 0)),
            pl.BlockSpec((1, H), lambda j: (0, 0)),
        ],
        out_specs=[
            pl.BlockSpec((R, H), lambda j: (j, 0)),
            pl.BlockSpec((R, H), lambda j: (j, 0)),
        ],
        out_shape=[
            jax.ShapeDtypeStruct((N, H), jnp.float32),
            jax.ShapeDtypeStruct((N, H), jnp.float32),
        ],
    )(b_z, W1n, W1s, b1row)


def _sc_edge1_body(y1n_hbm, src_hbm, dst_hbm, ew_hbm, s1_out, deg_out,
                   src_b, dst_b, ew_b, rows_b, ones_b, s1_sh, deg_sh, gsem):
    cid = lax.axis_index("c")
    sid = lax.axis_index("s")
    wid = sid * NC + cid
    ebase = wid * EPW

    one = jnp.ones((16,), jnp.float32)

    def ones_body(i, c):
        ones_b[pl.ds(i * 16, 16)] = one
        return c

    lax.fori_loop(0, C // 16, ones_body, 0)
    _zero_f32(ew_b, C)
    rows_flat = C * H

    def zrow_body(i, c):
        rows_b[i, pl.ds(0, 16)] = jnp.zeros((16,), jnp.float32)
        rows_b[i, pl.ds(16, 16)] = jnp.zeros((16,), jnp.float32)
        rows_b[i, pl.ds(32, 16)] = jnp.zeros((16,), jnp.float32)
        rows_b[i, pl.ds(48, 16)] = jnp.zeros((16,), jnp.float32)
        return c

    lax.fori_loop(0, C, zrow_body, 0)
    pltpu.sync_copy(rows_b, s1_sh.at[pl.ds(sid * ROWS_PT, C)])
    pltpu.sync_copy(rows_b.at[pl.ds(0, ROWS_PT - C)],
                    s1_sh.at[pl.ds(sid * ROWS_PT + C, ROWS_PT - C)])
    pltpu.sync_copy(ew_b, deg_sh.at[pl.ds(sid * ROWS_PT, C)])
    pltpu.sync_copy(ew_b.at[pl.ds(0, ROWS_PT - C)],
                    deg_sh.at[pl.ds(sid * ROWS_PT + C, ROWS_PT - C)])
    plsc.subcore_barrier()

    def chunk(it, carry):
        base = ebase + it * C
        for j in range(NSUB):
            pltpu.sync_copy(src_hbm.at[pl.ds(base + j * SUB, SUB)],
                            src_b.at[j])
            pltpu.sync_copy(dst_hbm.at[pl.ds(base + j * SUB, SUB)],
                            dst_b.at[j])
        pltpu.sync_copy(ew_hbm.at[pl.ds(base, C)], ew_b)
        for j in range(NSUB):
            pltpu.async_copy(y1n_hbm.at[src_b.at[j]],
                             rows_b.at[pl.ds(j * SUB, SUB)], gsem).wait()
        def scale(e, c):
            eb = jnp.full((16,), e, jnp.int32)
            wv = plsc.load_gather(ew_b, [eb])
            for k in range(4):
                v = rows_b[e, pl.ds(k * 16, 16)]
                rows_b[e, pl.ds(k * 16, 16)] = v * wv
            return c

        lax.fori_loop(0, C, scale, 0)
        for j in range(NSUB):
            pltpu.sync_copy(rows_b.at[pl.ds(j * SUB, SUB)],
                            s1_sh.at[dst_b.at[j]], add=True)
            pltpu.sync_copy(ones_b.at[pl.ds(j * SUB, SUB)],
                            deg_sh.at[dst_b.at[j]], add=True)
        return carry

    lax.fori_loop(0, NCHUNK, chunk, 0)
    plsc.subcore_barrier()

    rbase = sid * ROWS_PT

    @pl.when(rbase + ROWS_PT <= N)
    def _():
        pltpu.sync_copy(s1_sh.at[pl.ds(rbase, ROWS_PT)],
                        s1_out.at[cid, pl.ds(rbase, ROWS_PT)])

    @pl.when(rbase + ROWS_PT > N)
    def _():
        pltpu.sync_copy(s1_sh.at[pl.ds(rbase, N - 15 * ROWS_PT)],
                        s1_out.at[cid, pl.ds(rbase, N - 15 * ROWS_PT)])

    pltpu.sync_copy(deg_sh.at[pl.ds(rbase, ROWS_PT)],
                    deg_out.at[cid, pl.ds(rbase, ROWS_PT)])


def _sc_edge1(y1n, src, dst, ew):
    mesh = plsc.VectorSubcoreMesh(core_axis_name="c", subcore_axis_name="s")
    f = functools.partial(
        pl.kernel,
        compiler_params=pltpu.CompilerParams(needs_layout_passes=False, use_tc_tiling_on_sc=False),
        out_type=[
            jax.ShapeDtypeStruct((NC, N, H), jnp.float32),
            jax.ShapeDtypeStruct((NC, NPAD), jnp.float32),
        ],
        mesh=mesh,
        scratch_types=[
            pltpu.VMEM((NSUB, SUB), jnp.int32),
            pltpu.VMEM((NSUB, SUB), jnp.int32),
            pltpu.VMEM((C,), jnp.float32),
            pltpu.VMEM((C, H), jnp.float32),
            pltpu.VMEM((C,), jnp.float32),
            pltpu.VMEM_SHARED((NPAD, H), jnp.float32),
            pltpu.VMEM_SHARED((NPAD,), jnp.float32),
            pltpu.SemaphoreType.DMA,
        ],
    )(_sc_edge1_body)
    return f(y1n, src, dst, ew)


def _tc_mid_body(a1s_ref, s1a_ref, s1b_ref, deg_ref, w2n_ref, w2s_ref, b2_ref,
                 y2_ref, t2_ref):
    deg = deg_ref[...]
    denom = jnp.maximum(deg[:, 0:1] + deg[:, 1:2], 1.0)
    s1 = s1a_ref[...] + s1b_ref[...]
    h = jnp.tanh(a1s_ref[...] + s1 / denom)
    y2_ref[...] = jnp.sum(h * w2n_ref[...], axis=1, keepdims=True)
    t2_ref[...] = jnp.sum(h * w2s_ref[...], axis=1, keepdims=True) + b2_ref[...]


def _tc_mid(a1s, s1a, s1b, degT, W2nT, W2sT, b2sq):
    R = 1000
    return pl.pallas_call(
        _tc_mid_body,
        grid=(N // R,),
        in_specs=[
            pl.BlockSpec((R, H), lambda j: (j, 0)),
            pl.BlockSpec((R, H), lambda j: (j, 0)),
            pl.BlockSpec((R, H), lambda j: (j, 0)),
            pl.BlockSpec((R, NC), lambda j: (j, 0)),
            pl.BlockSpec((1, H), lambda j: (0, 0)),
            pl.BlockSpec((1, H), lambda j: (0, 0)),
            pl.BlockSpec((1, 1), lambda j: (0, 0)),
        ],
        out_specs=[
            pl.BlockSpec((R, 1), lambda j: (j, 0)),
            pl.BlockSpec((R, 1), lambda j: (j, 0)),
        ],
        out_shape=[
            jax.ShapeDtypeStruct((N, 1), jnp.float32),
            jax.ShapeDtypeStruct((N, 1), jnp.float32),
        ],
    )(a1s, s1a, s1b, degT, W2nT, W2sT, b2sq)


def _sc_edge2_body(y2_hbm, src_hbm, dst_hbm, ew_hbm, s2_out,
                   y2_b, src_b, dst_b, ew_b, m_b, s2_sh, gsem):
    cid = lax.axis_index("c")
    sid = lax.axis_index("s")
    wid = sid * NC + cid
    ebase = wid * EPW

    pltpu.sync_copy(y2_hbm, y2_b)
    _zero_f32(m_b, C)
    pltpu.sync_copy(m_b, s2_sh.at[pl.ds(sid * ROWS_PT, C)])
    pltpu.sync_copy(m_b.at[pl.ds(0, ROWS_PT - C)],
                    s2_sh.at[pl.ds(sid * ROWS_PT + C, ROWS_PT - C)])
    plsc.subcore_barrier()

    def chunk(it, carry):
        base = ebase + it * C
        pltpu.sync_copy(src_hbm.at[pl.ds(base, C)], src_b)
        pltpu.sync_copy(ew_hbm.at[pl.ds(base, C)], ew_b)
        for j in range(NSUB):
            pltpu.sync_copy(dst_hbm.at[pl.ds(base + j * SUB, SUB)],
                            dst_b.at[j])

        def grp(g, c):
            s16 = src_b[pl.ds(g * 16, 16)]
            vals = plsc.load_gather(y2_b, [s16])
            w16 = ew_b[pl.ds(g * 16, 16)]
            m_b[pl.ds(g * 16, 16)] = vals * w16
            return c

        lax.fori_loop(0, C // 16, grp, 0)
        for j in range(NSUB):
            pltpu.sync_copy(m_b.at[pl.ds(j * SUB, SUB)],
                            s2_sh.at[dst_b.at[j]], add=True)
        return carry

    lax.fori_loop(0, NCHUNK, chunk, 0)
    plsc.subcore_barrier()
    pltpu.sync_copy(s2_sh.at[pl.ds(sid * ROWS_PT, ROWS_PT)],
                    s2_out.at[cid, pl.ds(sid * ROWS_PT, ROWS_PT)])


def _sc_edge2(y2flat, src, dst, ew):
    mesh = plsc.VectorSubcoreMesh(core_axis_name="c", subcore_axis_name="s")
    f = functools.partial(
        pl.kernel,
        compiler_params=pltpu.CompilerParams(needs_layout_passes=False, use_tc_tiling_on_sc=False),
        out_type=[jax.ShapeDtypeStruct((NC, NPAD), jnp.float32)],
        mesh=mesh,
        scratch_types=[
            pltpu.VMEM((N,), jnp.float32),
            pltpu.VMEM((C,), jnp.int32),
            pltpu.VMEM((NSUB, SUB), jnp.int32),
            pltpu.VMEM((C,), jnp.float32),
            pltpu.VMEM((C,), jnp.float32),
            pltpu.VMEM_SHARED((NPAD,), jnp.float32),
            pltpu.SemaphoreType.DMA,
        ],
    )(_sc_edge2_body)
    return f(y2flat, src, dst, ew)[0]


def _tc_post_body(t2_ref, s2_ref, deg_ref, out_ref):
    deg = deg_ref[...]
    denom = jnp.maximum(deg[:, 0:1] + deg[:, 1:2], 1.0)
    s2 = s2_ref[...]
    out_ref[...] = t2_ref[...] + (s2[:, 0:1] + s2[:, 1:2]) / denom


def _tc_post(t2, s2T, degT):
    return pl.pallas_call(
        _tc_post_body,
        grid=(1,),
        in_specs=[
            pl.BlockSpec((N, 1), lambda j: (0, 0)),
            pl.BlockSpec((N, NC), lambda j: (0, 0)),
            pl.BlockSpec((N, NC), lambda j: (0, 0)),
        ],
        out_specs=pl.BlockSpec((N, 1), lambda j: (0, 0)),
        out_shape=jax.ShapeDtypeStruct((N, 1), jnp.float32),
    )(t2, s2T, degT)


def kernel(b_z, edge_index, b_edge_weight, b_size, W1s, W1n, b1, W2s, W2n, b2):
    src = edge_index[0]
    dst = edge_index[1]
    ew = b_edge_weight

    y1n, a1s = _tc_pre(b_z, W1n, W1s, b1.reshape(1, H))
    s1p, degp = _sc_edge1(y1n, src, dst, ew)
    degT = degp.T
    y2, t2 = _tc_mid(a1s, s1p[0], s1p[1], degT,
                     W2n.reshape(1, H), W2s.reshape(1, H), b2.reshape(1, 1))
    s2p = _sc_edge2(y2.reshape(N), src, dst, ew)
    out = _tc_post(t2, s2p.T, degT)
    return out.reshape(100, 100)


# speedup vs baseline: 14.5959x; 2.2562x over previous
import functools

import jax
import jax.numpy as jnp
from jax import lax
from jax.experimental import pallas as pl
from jax.experimental.pallas import tpu as pltpu
from jax.experimental.pallas import tpu_sc as plsc

N = 10000
E = 320000
D = 128
H = 64

NC = 2
NS = 16
NW = NC * NS
EPW = E // NW
C = 400
NSUB = 5
SUB = C // NSUB
NCHUNK = EPW // C
ROWS_PT = 640
NPAD = NS * ROWS_PT
DPR = EPW // SUB

_SC_PARAMS = pltpu.CompilerParams(
    needs_layout_passes=False, use_tc_tiling_on_sc=False)


def _tc_pre_body(bz_ref, w1n_ref, w1s_ref, b1_ref, y1n_ref, a1s_ref):
    x = bz_ref[...]
    y1n_ref[...] = jnp.dot(x, w1n_ref[...], preferred_element_type=jnp.float32)
    a1s_ref[...] = (
        jnp.dot(x, w1s_ref[...], preferred_element_type=jnp.float32)
        + b1_ref[...]
    )


def _tc_pre(b_z, W1n, W1s, b1row):
    R = 1000
    return pl.pallas_call(
        _tc_pre_body,
        grid=(N // R,),
        in_specs=[
            pl.BlockSpec((R, D), lambda j: (j, 0)),
            pl.BlockSpec((D, H), lambda j: (0, 0)),
            pl.BlockSpec((D, H), lambda j: (0, 0)),
            pl.BlockSpec((1, H), lambda j: (0, 0)),
        ],
        out_specs=[
            pl.BlockSpec((R, H), lambda j: (j, 0)),
            pl.BlockSpec((R, H), lambda j: (j, 0)),
        ],
        out_shape=[
            jax.ShapeDtypeStruct((N, H), jnp.float32),
            jax.ShapeDtypeStruct((N, H), jnp.float32),
        ],
    )(b_z, W1n, W1s, b1row)


def _sc_edge1_body(y1n_hbm, src_hbm, dst2_hbm, ew_hbm, s1_out, deg_out,
                   se0, se1, se2, de0, de1, de2, we0, we1, we2,
                   r0, r1, deg_acc, s1_sh,
                   ee0, ee1, ee2, g0, g1, s0, s1sem):
    cid = lax.axis_index("c")
    sid = lax.axis_index("s")
    wid = sid * NC + cid
    ebase = wid * EPW
    src_e = [se0, se1, se2]
    dst_e = [de0, de1, de2]
    ew_e = [we0, we1, we2]
    esems = [ee0, ee1, ee2]
    rows = [r0, r1]
    gsems = [g0, g1]
    ssems = [s0, s1sem]
    z16 = jnp.zeros((16,), jnp.float32)
    one16 = jnp.ones((16,), jnp.float32)

    def stage(m):
        b = m % 3
        return [
            pltpu.async_copy(src_hbm.at[pl.ds(ebase + m * C, C)],
                             src_e[b], esems[b]),
            pltpu.async_copy(dst2_hbm.at[pl.ds(wid * DPR + m * NSUB, NSUB)],
                             dst_e[b], esems[b]),
            pltpu.async_copy(ew_hbm.at[pl.ds(ebase + m * C, C)],
                             ew_e[b], esems[b]),
        ]

    def gather(m):
        b = m % 2
        return [
            pltpu.async_copy(
                y1n_hbm.at[src_e[m % 3].at[pl.ds(j * SUB, SUB)]],
                rows[b].at[pl.ds(j * SUB, SUB)], gsems[b])
            for j in range(NSUB)
        ]

    def scatter(m):
        b = m % 2
        return [
            pltpu.async_copy(rows[b].at[pl.ds(j * SUB, SUB)],
                             s1_sh.at[dst_e[m % 3].at[j]], ssems[b],
                             add=True)
            for j in range(NSUB)
        ]

    ed = {0: stage(0), 1: stage(1)}

    def zdeg(i, c):
        deg_acc[pl.ds(i * 64, 16)] = z16
        deg_acc[pl.ds(i * 64 + 16, 16)] = z16
        deg_acc[pl.ds(i * 64 + 32, 16)] = z16
        deg_acc[pl.ds(i * 64 + 48, 16)] = z16
        return c

    lax.fori_loop(0, NPAD // 64, zdeg, 0)

    def zrow(i, c):
        for r in range(4):
            for k in range(4):
                r0[i * 4 + r, pl.ds(k * 16, 16)] = z16
        return c

    lax.fori_loop(0, C // 4, zrow, 0)
    pltpu.sync_copy(r0, s1_sh.at[pl.ds(sid * ROWS_PT, C)])
    pltpu.sync_copy(r0.at[pl.ds(0, ROWS_PT - C)],
                    s1_sh.at[pl.ds(sid * ROWS_PT + C, ROWS_PT - C)])
    for d in ed.pop(0):
        d.wait()
    gd = {0: gather(0)}
    plsc.subcore_barrier()

    sd = {}
    for m in range(NCHUNK):
        b = m % 2
        eb = m % 3
        if m >= 1:
            for d in sd.pop(m - 1):
                d.wait()
        if m + 2 < NCHUNK:
            ed[m + 2] = stage(m + 2)
        if m + 1 < NCHUNK:
            for d in ed.pop(m + 1):
                d.wait()
            gd[m + 1] = gather(m + 1)
        for d in gd.pop(m):
            d.wait()

        def scale(e, c):
            eb16 = jnp.full((16,), e, jnp.int32)
            wv = plsc.load_gather(ew_e[eb], [eb16])
            for k in range(4):
                v = rows[b][e, pl.ds(k * 16, 16)]
                rows[b][e, pl.ds(k * 16, 16)] = v * wv
            return c

        lax.fori_loop(0, C, scale, 0)
        sd[m] = scatter(m)

        def deg(q, c):
            d16 = dst_e[eb][q // 5, pl.ds((q % 5) * 16, 16)]
            plsc.addupdate_scatter(deg_acc, [d16], one16)
            return c

        lax.fori_loop(0, C // 16, deg, 0)

    for m in sorted(sd):
        for d in sd[m]:
            d.wait()
    plsc.subcore_barrier()

    rbase = sid * ROWS_PT

    @pl.when(rbase + ROWS_PT <= N)
    def _():
        pltpu.sync_copy(s1_sh.at[pl.ds(rbase, ROWS_PT)],
                        s1_out.at[cid, pl.ds(rbase, ROWS_PT)])

    @pl.when(rbase + ROWS_PT > N)
    def _():
        pltpu.sync_copy(s1_sh.at[pl.ds(rbase, N - 15 * ROWS_PT)],
                        s1_out.at[cid, pl.ds(rbase, N - 15 * ROWS_PT)])

    pltpu.sync_copy(deg_acc, deg_out.at[wid])


def _sc_edge1(y1n, src, dst2d, ew):
    mesh = plsc.VectorSubcoreMesh(core_axis_name="c", subcore_axis_name="s")
    f = functools.partial(
        pl.kernel,
        compiler_params=_SC_PARAMS,
        out_type=[
            jax.ShapeDtypeStruct((NC, N, H), jnp.float32),
            jax.ShapeDtypeStruct((NW, NPAD), jnp.float32),
        ],
        mesh=mesh,
        scratch_types=(
            [pltpu.VMEM((C,), jnp.int32)] * 3
            + [pltpu.VMEM((NSUB, SUB), jnp.int32)] * 3
            + [pltpu.VMEM((C,), jnp.float32)] * 3
            + [
                pltpu.VMEM((C, H), jnp.float32),
                pltpu.VMEM((C, H), jnp.float32),
                pltpu.VMEM((NPAD,), jnp.float32),
                pltpu.VMEM_SHARED((NPAD, H), jnp.float32),
            ]
            + [pltpu.SemaphoreType.DMA] * 7
        ),
    )(_sc_edge1_body)
    return f(y1n, src, dst2d, ew)


def _tc_mid_body(a1s_ref, s1a_ref, s1b_ref, deg_ref, w2n_ref, w2s_ref, b2_ref,
                 y2_ref, t2_ref, dn_ref):
    denom = jnp.maximum(jnp.sum(deg_ref[...], axis=1, keepdims=True), 1.0)
    s1 = s1a_ref[...] + s1b_ref[...]
    h = jnp.tanh(a1s_ref[...] + s1 / denom)
    y2_ref[...] = jnp.sum(h * w2n_ref[...], axis=1, keepdims=True)
    t2_ref[...] = jnp.sum(h * w2s_ref[...], axis=1, keepdims=True) + b2_ref[...]
    dn_ref[...] = denom


def _tc_mid(a1s, s1a, s1b, degT, W2nT, W2sT, b2sq):
    R = 1000
    return pl.pallas_call(
        _tc_mid_body,
        grid=(N // R,),
        in_specs=[
            pl.BlockSpec((R, H), lambda j: (j, 0)),
            pl.BlockSpec((R, H), lambda j: (j, 0)),
            pl.BlockSpec((R, H), lambda j: (j, 0)),
            pl.BlockSpec((R, NW), lambda j: (j, 0)),
            pl.BlockSpec((1, H), lambda j: (0, 0)),
            pl.BlockSpec((1, H), lambda j: (0, 0)),
            pl.BlockSpec((1, 1), lambda j: (0, 0)),
        ],
        out_specs=[
            pl.BlockSpec((R, 1), lambda j: (j, 0)),
            pl.BlockSpec((R, 1), lambda j: (j, 0)),
            pl.BlockSpec((R, 1), lambda j: (j, 0)),
        ],
        out_shape=[
            jax.ShapeDtypeStruct((N, 1), jnp.float32),
            jax.ShapeDtypeStruct((N, 1), jnp.float32),
            jax.ShapeDtypeStruct((N, 1), jnp.float32),
        ],
    )(a1s, s1a, s1b, degT, W2nT, W2sT, b2sq)


def _sc_edge2_body(y2_hbm, src_hbm, dst_hbm, ew_hbm, s2_out,
                   y2_b, src_b, dst_b, ew_b, acc, esem):
    cid = lax.axis_index("c")
    sid = lax.axis_index("s")
    wid = sid * NC + cid
    ebase = wid * EPW
    z16 = jnp.zeros((16,), jnp.float32)

    e0 = pltpu.async_copy(y2_hbm, y2_b, esem)
    e1 = pltpu.async_copy(src_hbm.at[pl.ds(ebase, EPW)], src_b, esem)
    e2 = pltpu.async_copy(dst_hbm.at[pl.ds(ebase, EPW)], dst_b, esem)
    e3 = pltpu.async_copy(ew_hbm.at[pl.ds(ebase, EPW)], ew_b, esem)

    def zacc(i, c):
        acc[pl.ds(i * 64, 16)] = z16
        acc[pl.ds(i * 64 + 16, 16)] = z16
        acc[pl.ds(i * 64 + 32, 16)] = z16
        acc[pl.ds(i * 64 + 48, 16)] = z16
        return c

    lax.fori_loop(0, NPAD // 64, zacc, 0)
    e0.wait()
    e1.wait()
    e2.wait()
    e3.wait()

    def grp(g, c):
        s16 = src_b[pl.ds(g * 16, 16)]
        d16 = dst_b[pl.ds(g * 16, 16)]
        vals = plsc.load_gather(y2_b, [s16])
        w16 = ew_b[pl.ds(g * 16, 16)]
        plsc.addupdate_scatter(acc, [d16], vals * w16)
        return c

    lax.fori_loop(0, EPW // 16, grp, 0)
    pltpu.sync_copy(acc, s2_out.at[wid])


def _sc_edge2(y2flat, src, dst, ew):
    mesh = plsc.VectorSubcoreMesh(core_axis_name="c", subcore_axis_name="s")
    f = functools.partial(
        pl.kernel,
        compiler_params=_SC_PARAMS,
        out_type=[jax.ShapeDtypeStruct((NW, NPAD), jnp.float32)],
        mesh=mesh,
        scratch_types=[
            pltpu.VMEM((N,), jnp.float32),
            pltpu.VMEM((EPW,), jnp.int32),
            pltpu.VMEM((EPW,), jnp.int32),
            pltpu.VMEM((EPW,), jnp.float32),
            pltpu.VMEM((NPAD,), jnp.float32),
            pltpu.SemaphoreType.DMA,
        ],
    )(_sc_edge2_body)
    return f(y2flat, src, dst, ew)[0]


def _tc_post_body(t2_ref, s2_ref, dn_ref, out_ref):
    s2 = jnp.sum(s2_ref[...], axis=1, keepdims=True)
    out_ref[...] = t2_ref[...] + s2 / dn_ref[...]


def _tc_post(t2, s2T, dn):
    return pl.pallas_call(
        _tc_post_body,
        grid=(1,),
        in_specs=[
            pl.BlockSpec((N, 1), lambda j: (0, 0)),
            pl.BlockSpec((N, NW), lambda j: (0, 0)),
            pl.BlockSpec((N, 1), lambda j: (0, 0)),
        ],
        out_specs=pl.BlockSpec((N, 1), lambda j: (0, 0)),
        out_shape=jax.ShapeDtypeStruct((N, 1), jnp.float32),
    )(t2, s2T, dn)


def kernel(b_z, edge_index, b_edge_weight, b_size, W1s, W1n, b1, W2s, W2n, b2):
    src = edge_index[0]
    dst = edge_index[1]
    ew = b_edge_weight
    dst2d = dst.reshape(E // SUB, SUB)

    y1n, a1s = _tc_pre(b_z, W1n, W1s, b1.reshape(1, H))
    s1p, degp = _sc_edge1(y1n, src, dst2d, ew)
    degT = degp.T
    y2, t2, dn = _tc_mid(a1s, s1p[0], s1p[1], degT,
                         W2n.reshape(1, H), W2s.reshape(1, H),
                         b2.reshape(1, 1))
    s2p = _sc_edge2(y2.reshape(N), src, dst, ew)
    out = _tc_post(t2, s2p.T, dn)
    return out.reshape(100, 100)


# speedup vs baseline: 16.7513x; 1.1477x over previous
import functools

import jax
import jax.numpy as jnp
from jax import lax
from jax.experimental import pallas as pl
from jax.experimental.pallas import tpu as pltpu
from jax.experimental.pallas import tpu_sc as plsc

N = 10000
E = 320000
D = 128
H = 64

NC = 2
NS = 16
NW = NC * NS
EPW = E // NW
C = 400
NSUB = 5
SUB = C // NSUB
NCHUNK = EPW // C
ROWS_PT = 640
NPAD = NS * ROWS_PT
DPR = EPW // SUB

_SC_PARAMS = pltpu.CompilerParams(
    needs_layout_passes=False, use_tc_tiling_on_sc=False)


def _tc_pre_body(bz_ref, w1n_ref, w1s_ref, b1_ref, y1n_ref, a1s_ref):
    x = bz_ref[...]
    y1n_ref[...] = jnp.dot(x, w1n_ref[...], preferred_element_type=jnp.float32)
    a1s_ref[...] = (
        jnp.dot(x, w1s_ref[...], preferred_element_type=jnp.float32)
        + b1_ref[...]
    )


def _tc_pre(b_z, W1n, W1s, b1):
    R = 1000
    return pl.pallas_call(
        _tc_pre_body,
        grid=(N // R,),
        in_specs=[
            pl.BlockSpec((R, D), lambda j: (j, 0)),
            pl.BlockSpec((D, H), lambda j: (0, 0)),
            pl.BlockSpec((D, H), lambda j: (0, 0)),
            pl.BlockSpec((H,), lambda j: (0,)),
        ],
        out_specs=[
            pl.BlockSpec((R, H), lambda j: (j, 0)),
            pl.BlockSpec((R, H), lambda j: (j, 0)),
        ],
        out_shape=[
            jax.ShapeDtypeStruct((N, H), jnp.float32),
            jax.ShapeDtypeStruct((N, H), jnp.float32),
        ],
    )(b_z, W1n, W1s, b1)


def _sc_edge1_body(y1n_hbm, ei_hbm, dst2_hbm, ew_hbm, s1_out, deg_out,
                   se0, se1, se2, de0, de1, de2, we0, we1, we2,
                   r0, r1, deg_acc, s1_sh,
                   ee0, ee1, ee2, g0, g1, s0, s1sem):
    cid = lax.axis_index("c")
    sid = lax.axis_index("s")
    wid = sid * NC + cid
    ebase = wid * EPW
    src_e = [se0, se1, se2]
    dst_e = [de0, de1, de2]
    ew_e = [we0, we1, we2]
    esems = [ee0, ee1, ee2]
    rows = [r0, r1]
    gsems = [g0, g1]
    ssems = [s0, s1sem]
    z16 = jnp.zeros((16,), jnp.float32)
    one16 = jnp.ones((16,), jnp.float32)

    def stage(m):
        b = m % 3
        return [
            pltpu.async_copy(ei_hbm.at[0, pl.ds(ebase + m * C, C)],
                             src_e[b], esems[b]),
            pltpu.async_copy(dst2_hbm.at[pl.ds(wid * DPR + m * NSUB, NSUB)],
                             dst_e[b], esems[b]),
            pltpu.async_copy(ew_hbm.at[pl.ds(ebase + m * C, C)],
                             ew_e[b], esems[b]),
        ]

    def gather(m):
        b = m % 2
        return [
            pltpu.async_copy(
                y1n_hbm.at[src_e[m % 3].at[pl.ds(j * SUB, SUB)]],
                rows[b].at[pl.ds(j * SUB, SUB)], gsems[b])
            for j in range(NSUB)
        ]

    def scatter(m):
        b = m % 2
        return [
            pltpu.async_copy(rows[b].at[pl.ds(j * SUB, SUB)],
                             s1_sh.at[dst_e[m % 3].at[j]], ssems[b],
                             add=True)
            for j in range(NSUB)
        ]

    ed = {0: stage(0), 1: stage(1)}

    def zdeg(i, c):
        deg_acc[pl.ds(i * 64, 16)] = z16
        deg_acc[pl.ds(i * 64 + 16, 16)] = z16
        deg_acc[pl.ds(i * 64 + 32, 16)] = z16
        deg_acc[pl.ds(i * 64 + 48, 16)] = z16
        return c

    lax.fori_loop(0, NPAD // 64, zdeg, 0)

    def zrow(i, c):
        for r in range(4):
            for k in range(4):
                r0[i * 4 + r, pl.ds(k * 16, 16)] = z16
        return c

    lax.fori_loop(0, C // 4, zrow, 0)
    pltpu.sync_copy(r0, s1_sh.at[pl.ds(sid * ROWS_PT, C)])
    pltpu.sync_copy(r0.at[pl.ds(0, ROWS_PT - C)],
                    s1_sh.at[pl.ds(sid * ROWS_PT + C, ROWS_PT - C)])
    for d in ed.pop(0):
        d.wait()
    gd = {0: gather(0)}
    plsc.subcore_barrier()

    sd = {}
    for m in range(NCHUNK):
        b = m % 2
        eb = m % 3
        if m >= 1:
            for d in sd.pop(m - 1):
                d.wait()
        if m + 2 < NCHUNK:
            ed[m + 2] = stage(m + 2)
        if m + 1 < NCHUNK:
            for d in ed.pop(m + 1):
                d.wait()
            gd[m + 1] = gather(m + 1)
        for d in gd.pop(m):
            d.wait()

        @plsc.parallel_loop(0, C, unroll=4)
        def scale(e):
            eb16 = jnp.full((16,), e, jnp.int32)
            wv = plsc.load_gather(ew_e[eb], [eb16])
            for k in range(4):
                v = rows[b][e, pl.ds(k * 16, 16)]
                rows[b][e, pl.ds(k * 16, 16)] = v * wv

        sd[m] = scatter(m)

        for jj in range(NSUB):
            def deg(q, c, _jj=jj):
                d16 = dst_e[eb][_jj, pl.ds(q * 16, 16)]
                plsc.addupdate_scatter(deg_acc, [d16], one16)
                return c

            lax.fori_loop(0, SUB // 16, deg, 0, unroll=5)

    for m in sorted(sd):
        for d in sd[m]:
            d.wait()
    plsc.subcore_barrier()

    rbase = sid * ROWS_PT

    @pl.when(rbase + ROWS_PT <= N)
    def _():
        pltpu.sync_copy(s1_sh.at[pl.ds(rbase, ROWS_PT)],
                        s1_out.at[cid, pl.ds(rbase, ROWS_PT)])

    @pl.when(rbase + ROWS_PT > N)
    def _():
        pltpu.sync_copy(s1_sh.at[pl.ds(rbase, N - 15 * ROWS_PT)],
                        s1_out.at[cid, pl.ds(rbase, N - 15 * ROWS_PT)])

    pltpu.sync_copy(deg_acc, deg_out.at[wid])


def _sc_edge1(y1n, ei, dst2d, ew):
    mesh = plsc.VectorSubcoreMesh(core_axis_name="c", subcore_axis_name="s")
    f = functools.partial(
        pl.kernel,
        compiler_params=_SC_PARAMS,
        out_type=[
            jax.ShapeDtypeStruct((NC, N, H), jnp.float32),
            jax.ShapeDtypeStruct((NW, NPAD), jnp.float32),
        ],
        mesh=mesh,
        scratch_types=(
            [pltpu.VMEM((C,), jnp.int32)] * 3
            + [pltpu.VMEM((NSUB, SUB), jnp.int32)] * 3
            + [pltpu.VMEM((C,), jnp.float32)] * 3
            + [
                pltpu.VMEM((C, H), jnp.float32),
                pltpu.VMEM((C, H), jnp.float32),
                pltpu.VMEM((NPAD,), jnp.float32),
                pltpu.VMEM_SHARED((NPAD, H), jnp.float32),
            ]
            + [pltpu.SemaphoreType.DMA] * 7
        ),
    )(_sc_edge1_body)
    return f(y1n, ei, dst2d, ew)


def _tc_mid_body(a1s_ref, s1a_ref, s1b_ref, deg_ref, w2n_ref, w2s_ref, b2_ref,
                 y2_ref, t2_ref, dn_ref):
    denom = jnp.maximum(jnp.sum(deg_ref[...], axis=1, keepdims=True), 1.0)
    s1 = s1a_ref[...] + s1b_ref[...]
    h = jnp.tanh(a1s_ref[...] + s1 / denom)
    y2_ref[...] = jnp.dot(h, w2n_ref[...], preferred_element_type=jnp.float32)
    t2_ref[...] = (
        jnp.dot(h, w2s_ref[...], preferred_element_type=jnp.float32)
        + b2_ref[...]
    )
    dn_ref[...] = denom


def _tc_mid(a1s, s1a, s1b, degT, W2n, W2s, b2):
    R = 1000
    return pl.pallas_call(
        _tc_mid_body,
        grid=(N // R,),
        in_specs=[
            pl.BlockSpec((R, H), lambda j: (j, 0)),
            pl.BlockSpec((R, H), lambda j: (j, 0)),
            pl.BlockSpec((R, H), lambda j: (j, 0)),
            pl.BlockSpec((R, NW), lambda j: (j, 0)),
            pl.BlockSpec((H, 1), lambda j: (0, 0)),
            pl.BlockSpec((H, 1), lambda j: (0, 0)),
            pl.BlockSpec((1,), lambda j: (0,)),
        ],
        out_specs=[
            pl.BlockSpec((R, 1), lambda j: (j, 0)),
            pl.BlockSpec((R, 1), lambda j: (j, 0)),
            pl.BlockSpec((R, 1), lambda j: (j, 0)),
        ],
        out_shape=[
            jax.ShapeDtypeStruct((N, 1), jnp.float32),
            jax.ShapeDtypeStruct((N, 1), jnp.float32),
            jax.ShapeDtypeStruct((N, 1), jnp.float32),
        ],
    )(a1s, s1a, s1b, degT, W2n, W2s, b2)


def _sc_edge2_body(y2_hbm, ei_hbm, ew_hbm, s2_out,
                   y2_b, src_b, dst_b, ew_b, acc, esem):
    cid = lax.axis_index("c")
    sid = lax.axis_index("s")
    wid = sid * NC + cid
    ebase = wid * EPW
    z16 = jnp.zeros((16,), jnp.float32)

    e0 = pltpu.async_copy(y2_hbm, y2_b, esem)
    e1 = pltpu.async_copy(ei_hbm.at[0, pl.ds(ebase, EPW)], src_b, esem)
    e2 = pltpu.async_copy(ei_hbm.at[1, pl.ds(ebase, EPW)], dst_b, esem)
    e3 = pltpu.async_copy(ew_hbm.at[pl.ds(ebase, EPW)], ew_b, esem)

    def zacc(i, c):
        acc[pl.ds(i * 64, 16)] = z16
        acc[pl.ds(i * 64 + 16, 16)] = z16
        acc[pl.ds(i * 64 + 32, 16)] = z16
        acc[pl.ds(i * 64 + 48, 16)] = z16
        return c

    lax.fori_loop(0, NPAD // 64, zacc, 0)
    e0.wait()
    e1.wait()
    e2.wait()
    e3.wait()

    zi16 = jnp.zeros((16,), jnp.int32)

    def grp(g, c):
        s16 = src_b[pl.ds(g * 16, 16)]
        d16 = dst_b[pl.ds(g * 16, 16)]
        vals = plsc.load_gather(y2_b, [s16, zi16])
        w16 = ew_b[pl.ds(g * 16, 16)]
        plsc.addupdate_scatter(acc, [d16], vals * w16)
        return c

    lax.fori_loop(0, EPW // 16, grp, 0, unroll=4)
    pltpu.sync_copy(acc, s2_out.at[wid])


def _sc_edge2(y2, ei, ew):
    mesh = plsc.VectorSubcoreMesh(core_axis_name="c", subcore_axis_name="s")
    f = functools.partial(
        pl.kernel,
        compiler_params=_SC_PARAMS,
        out_type=[jax.ShapeDtypeStruct((NW, NPAD), jnp.float32)],
        mesh=mesh,
        scratch_types=[
            pltpu.VMEM((N, 1), jnp.float32),
            pltpu.VMEM((EPW,), jnp.int32),
            pltpu.VMEM((EPW,), jnp.int32),
            pltpu.VMEM((EPW,), jnp.float32),
            pltpu.VMEM((NPAD,), jnp.float32),
            pltpu.SemaphoreType.DMA,
        ],
    )(_sc_edge2_body)
    return f(y2, ei, ew)[0]


def _tc_post_body(t2_ref, s2_ref, dn_ref, out_ref):
    s2 = jnp.sum(s2_ref[...], axis=1, keepdims=True)
    out_ref[...] = t2_ref[...] + s2 / dn_ref[...]


def _tc_post(t2, s2T, dn):
    return pl.pallas_call(
        _tc_post_body,
        grid=(1,),
        in_specs=[
            pl.BlockSpec((N, 1), lambda j: (0, 0)),
            pl.BlockSpec((N, NW), lambda j: (0, 0)),
            pl.BlockSpec((N, 1), lambda j: (0, 0)),
        ],
        out_specs=pl.BlockSpec((N, 1), lambda j: (0, 0)),
        out_shape=jax.ShapeDtypeStruct((N, 1), jnp.float32),
    )(t2, s2T, dn)


def kernel(b_z, edge_index, b_edge_weight, b_size, W1s, W1n, b1, W2s, W2n, b2):
    ew = b_edge_weight
    dst2d = edge_index[1].reshape(E // SUB, SUB)

    y1n, a1s = _tc_pre(b_z, W1n, W1s, b1)
    s1p, degp = _sc_edge1(y1n, edge_index, dst2d, ew)
    degT = degp.T
    y2, t2, dn = _tc_mid(a1s, s1p[0], s1p[1], degT, W2n, W2s, b2)
    s2p = _sc_edge2(y2, edge_index, ew)
    out = _tc_post(t2, s2p.T, dn)
    return out.reshape(100, 100)


# speedup vs baseline: 17.0530x; 1.0180x over previous
import functools

import jax
import jax.numpy as jnp
from jax import lax
from jax.experimental import pallas as pl
from jax.experimental.pallas import tpu as pltpu
from jax.experimental.pallas import tpu_sc as plsc

N = 10000
E = 320000
D = 128
H = 64

NC = 2
NS = 16
NW = NC * NS
EPW = E // NW
C = 400
NSUB = 5
SUB = C // NSUB
NCHUNK = EPW // C
ROWS_PT = 640
NPAD = NS * ROWS_PT
DPR = EPW // SUB

_SC_PARAMS = pltpu.CompilerParams(
    needs_layout_passes=False, use_tc_tiling_on_sc=False)


def _tc_pre_body(bz_ref, w1n_ref, w1s_ref, b1_ref, y1n_ref, a1s_ref):
    x = bz_ref[...]
    y1n_ref[...] = jnp.dot(x, w1n_ref[...], preferred_element_type=jnp.float32)
    a1s_ref[...] = (
        jnp.dot(x, w1s_ref[...], preferred_element_type=jnp.float32)
        + b1_ref[...]
    )


def _tc_pre(b_z, W1n, W1s, b1):
    R = 1000
    return pl.pallas_call(
        _tc_pre_body,
        grid=(N // R,),
        in_specs=[
            pl.BlockSpec((R, D), lambda j: (j, 0)),
            pl.BlockSpec((D, H), lambda j: (0, 0)),
            pl.BlockSpec((D, H), lambda j: (0, 0)),
            pl.BlockSpec((H,), lambda j: (0,)),
        ],
        out_specs=[
            pl.BlockSpec((R, H), lambda j: (j, 0)),
            pl.BlockSpec((R, H), lambda j: (j, 0)),
        ],
        out_shape=[
            jax.ShapeDtypeStruct((N, H), jnp.float32),
            jax.ShapeDtypeStruct((N, H), jnp.float32),
        ],
    )(b_z, W1n, W1s, b1)


def _sc_edge1_body(y1n_hbm, ei_hbm, dst2_hbm, ew_hbm, zr_hbm, zn_hbm,
                   s1_out, deg_out,
                   se0, se1, se2, de0, de1, de2, we0, we1, we2,
                   r0, r1, deg_acc, s1_sh,
                   ee0, ee1, ee2, g0, g1, s0, s1sem):
    cid = lax.axis_index("c")
    sid = lax.axis_index("s")
    wid = sid * NC + cid
    ebase = wid * EPW
    src_e = [se0, se1, se2]
    dst_e = [de0, de1, de2]
    ew_e = [we0, we1, we2]
    esems = [ee0, ee1, ee2]
    rows = [r0, r1]
    gsems = [g0, g1]
    ssems = [s0, s1sem]
    z16 = jnp.zeros((16,), jnp.float32)
    one16 = jnp.ones((16,), jnp.float32)

    def stage(m):
        b = m % 3
        return [
            pltpu.async_copy(ei_hbm.at[0, pl.ds(ebase + m * C, C)],
                             src_e[b], esems[b]),
            pltpu.async_copy(dst2_hbm.at[pl.ds(wid * DPR + m * NSUB, NSUB)],
                             dst_e[b], esems[b]),
            pltpu.async_copy(ew_hbm.at[pl.ds(ebase + m * C, C)],
                             ew_e[b], esems[b]),
        ]

    def gather(m):
        b = m % 2
        return [
            pltpu.async_copy(
                y1n_hbm.at[src_e[m % 3].at[pl.ds(j * SUB, SUB)]],
                rows[b].at[pl.ds(j * SUB, SUB)], gsems[b])
            for j in range(NSUB)
        ]

    def scatter(m):
        b = m % 2
        return [
            pltpu.async_copy(rows[b].at[pl.ds(j * SUB, SUB)],
                             s1_sh.at[dst_e[m % 3].at[j]], ssems[b],
                             add=True)
            for j in range(NSUB)
        ]

    ed = {0: stage(0), 1: stage(1)}
    z1 = pltpu.async_copy(zn_hbm, deg_acc, g0)
    z2 = pltpu.async_copy(zr_hbm, s1_sh.at[pl.ds(sid * ROWS_PT, ROWS_PT)],
                          g1)
    z1.wait()
    z2.wait()
    for d in ed.pop(0):
        d.wait()
    gd = {0: gather(0)}
    plsc.subcore_barrier()

    sd = {}
    for m in range(NCHUNK):
        b = m % 2
        eb = m % 3
        if m >= 1:
            for d in sd.pop(m - 1):
                d.wait()
        if m + 2 < NCHUNK:
            ed[m + 2] = stage(m + 2)
        if m + 1 < NCHUNK:
            for d in ed.pop(m + 1):
                d.wait()
            gd[m + 1] = gather(m + 1)
        for d in gd.pop(m):
            d.wait()

        @plsc.parallel_loop(0, C, unroll=4)
        def scale(e):
            eb16 = jnp.full((16,), e, jnp.int32)
            wv = plsc.load_gather(ew_e[eb], [eb16])
            for k in range(4):
                v = rows[b][e, pl.ds(k * 16, 16)]
                rows[b][e, pl.ds(k * 16, 16)] = v * wv

        sd[m] = scatter(m)

        for jj in range(NSUB):
            def deg(q, c, _jj=jj):
                d16 = dst_e[eb][_jj, pl.ds(q * 16, 16)]
                plsc.addupdate_scatter(deg_acc, [d16], one16)
                return c

            lax.fori_loop(0, SUB // 16, deg, 0, unroll=5)

    for m in sorted(sd):
        for d in sd[m]:
            d.wait()
    plsc.subcore_barrier()

    rbase = sid * ROWS_PT

    @pl.when(rbase + ROWS_PT <= N)
    def _():
        pltpu.sync_copy(s1_sh.at[pl.ds(rbase, ROWS_PT)],
                        s1_out.at[cid, pl.ds(rbase, ROWS_PT)])

    @pl.when(rbase + ROWS_PT > N)
    def _():
        pltpu.sync_copy(s1_sh.at[pl.ds(rbase, N - 15 * ROWS_PT)],
                        s1_out.at[cid, pl.ds(rbase, N - 15 * ROWS_PT)])

    pltpu.sync_copy(deg_acc, deg_out.at[wid])


def _sc_edge1(y1n, ei, dst2d, ew, zr, zn):
    mesh = plsc.VectorSubcoreMesh(core_axis_name="c", subcore_axis_name="s")
    f = functools.partial(
        pl.kernel,
        compiler_params=_SC_PARAMS,
        out_type=[
            jax.ShapeDtypeStruct((NC, N, H), jnp.float32),
            jax.ShapeDtypeStruct((NW, NPAD), jnp.float32),
        ],
        mesh=mesh,
        scratch_types=(
            [pltpu.VMEM((C,), jnp.int32)] * 3
            + [pltpu.VMEM((NSUB, SUB), jnp.int32)] * 3
            + [pltpu.VMEM((C,), jnp.float32)] * 3
            + [
                pltpu.VMEM((C, H), jnp.float32),
                pltpu.VMEM((C, H), jnp.float32),
                pltpu.VMEM((NPAD,), jnp.float32),
                pltpu.VMEM_SHARED((NPAD, H), jnp.float32),
            ]
            + [pltpu.SemaphoreType.DMA] * 7
        ),
    )(_sc_edge1_body)
    return f(y1n, ei, dst2d, ew, zr, zn)


def _tc_mid_body(a1s_ref, s1a_ref, s1b_ref, deg_ref, w2n_ref, w2s_ref, b2_ref,
                 y2_ref, t2_ref, dn_ref):
    denom = jnp.maximum(jnp.sum(deg_ref[...], axis=1, keepdims=True), 1.0)
    s1 = s1a_ref[...] + s1b_ref[...]
    h = jnp.tanh(a1s_ref[...] + s1 / denom)
    y2_ref[...] = jnp.dot(h, w2n_ref[...], preferred_element_type=jnp.float32)
    t2_ref[...] = (
        jnp.dot(h, w2s_ref[...], preferred_element_type=jnp.float32)
        + b2_ref[...]
    )
    dn_ref[...] = denom


def _tc_mid(a1s, s1a, s1b, degT, W2n, W2s, b2):
    R = 1000
    return pl.pallas_call(
        _tc_mid_body,
        grid=(N // R,),
        in_specs=[
            pl.BlockSpec((R, H), lambda j: (j, 0)),
            pl.BlockSpec((R, H), lambda j: (j, 0)),
            pl.BlockSpec((R, H), lambda j: (j, 0)),
            pl.BlockSpec((R, NW), lambda j: (j, 0)),
            pl.BlockSpec((H, 1), lambda j: (0, 0)),
            pl.BlockSpec((H, 1), lambda j: (0, 0)),
            pl.BlockSpec((1,), lambda j: (0,)),
        ],
        out_specs=[
            pl.BlockSpec((R, 1), lambda j: (j, 0)),
            pl.BlockSpec((R, 1), lambda j: (j, 0)),
            pl.BlockSpec((R, 1), lambda j: (j, 0)),
        ],
        out_shape=[
            jax.ShapeDtypeStruct((N, 1), jnp.float32),
            jax.ShapeDtypeStruct((N, 1), jnp.float32),
            jax.ShapeDtypeStruct((N, 1), jnp.float32),
        ],
    )(a1s, s1a, s1b, degT, W2n, W2s, b2)


def _sc_edge2_body(y2_hbm, ei_hbm, ew_hbm, zn_hbm, s2_out,
                   y2_b, src_b, dst_b, ew_b, acc, esem):
    cid = lax.axis_index("c")
    sid = lax.axis_index("s")
    wid = sid * NC + cid
    ebase = wid * EPW

    e0 = pltpu.async_copy(y2_hbm, y2_b, esem)
    e1 = pltpu.async_copy(ei_hbm.at[0, pl.ds(ebase, EPW)], src_b, esem)
    e2 = pltpu.async_copy(ei_hbm.at[1, pl.ds(ebase, EPW)], dst_b, esem)
    e3 = pltpu.async_copy(ew_hbm.at[pl.ds(ebase, EPW)], ew_b, esem)
    e4 = pltpu.async_copy(zn_hbm, acc, esem)
    e4.wait()
    e0.wait()
    e1.wait()
    e2.wait()
    e3.wait()

    def grp(g, c):
        s16 = src_b[pl.ds(g * 16, 16)]
        d16 = dst_b[pl.ds(g * 16, 16)]
        vals = plsc.load_gather(y2_b, [s16])
        w16 = ew_b[pl.ds(g * 16, 16)]
        plsc.addupdate_scatter(acc, [d16], vals * w16)
        return c

    lax.fori_loop(0, EPW // 16, grp, 0)
    pltpu.sync_copy(acc, s2_out.at[wid])


def _sc_edge2(y2, ei, ew, zn):
    mesh = plsc.VectorSubcoreMesh(core_axis_name="c", subcore_axis_name="s")
    f = functools.partial(
        pl.kernel,
        compiler_params=_SC_PARAMS,
        out_type=[jax.ShapeDtypeStruct((NW, NPAD), jnp.float32)],
        mesh=mesh,
        scratch_types=[
            pltpu.VMEM((N,), jnp.float32),
            pltpu.VMEM((EPW,), jnp.int32),
            pltpu.VMEM((EPW,), jnp.int32),
            pltpu.VMEM((EPW,), jnp.float32),
            pltpu.VMEM((NPAD,), jnp.float32),
            pltpu.SemaphoreType.DMA,
        ],
    )(_sc_edge2_body)
    return f(y2, ei, ew, zn)[0]


def _tc_post_body(t2_ref, s2_ref, dn_ref, out_ref):
    s2 = jnp.sum(s2_ref[...], axis=1, keepdims=True)
    out_ref[...] = t2_ref[...] + s2 / dn_ref[...]


def _tc_post(t2, s2T, dn):
    return pl.pallas_call(
        _tc_post_body,
        grid=(1,),
        in_specs=[
            pl.BlockSpec((N, 1), lambda j: (0, 0)),
            pl.BlockSpec((N, NW), lambda j: (0, 0)),
            pl.BlockSpec((N, 1), lambda j: (0, 0)),
        ],
        out_specs=pl.BlockSpec((N, 1), lambda j: (0, 0)),
        out_shape=jax.ShapeDtypeStruct((N, 1), jnp.float32),
    )(t2, s2T, dn)


def kernel(b_z, edge_index, b_edge_weight, b_size, W1s, W1n, b1, W2s, W2n, b2):
    ew = b_edge_weight
    dst2d = edge_index[1].reshape(E // SUB, SUB)

    zr = jnp.zeros((ROWS_PT, H), jnp.float32)
    zn = jnp.zeros((NPAD,), jnp.float32)
    y1n, a1s = _tc_pre(b_z, W1n, W1s, b1)
    s1p, degp = _sc_edge1(y1n, edge_index, dst2d, ew, zr, zn)
    degT = degp.T
    y2, t2, dn = _tc_mid(a1s, s1p[0], s1p[1], degT, W2n, W2s, b2)
    s2p = _sc_edge2(y2.reshape(N), edge_index, ew, zn)
    out = _tc_post(t2, s2p.T, dn)
    return out.reshape(100, 100)


# speedup vs baseline: 17.9995x; 1.0555x over previous
import functools

import jax
import jax.numpy as jnp
from jax import lax
from jax.experimental import pallas as pl
from jax.experimental.pallas import tpu as pltpu
from jax.experimental.pallas import tpu_sc as plsc

N = 10000
E = 320000
D = 128
H = 64

NC = 2
NS = 16
NW = NC * NS
EPW = E // NW
C = 400
NSUB = 5
SUB = C // NSUB
NCHUNK = EPW // C
ROWS_PT = 640
NPAD = NS * ROWS_PT
DPR = EPW // SUB

_SC_PARAMS = pltpu.CompilerParams(
    needs_layout_passes=False, use_tc_tiling_on_sc=False)


def _tc_pre_body(bz_ref, w1n_ref, y1n_ref):
    y1n_ref[...] = jnp.dot(bz_ref[...], w1n_ref[...],
                           preferred_element_type=jnp.float32)


def _tc_pre(b_z, W1n):
    R = 1000
    return pl.pallas_call(
        _tc_pre_body,
        grid=(N // R,),
        in_specs=[
            pl.BlockSpec((R, D), lambda j: (j, 0)),
            pl.BlockSpec((D, H), lambda j: (0, 0)),
        ],
        out_specs=pl.BlockSpec((R, H), lambda j: (j, 0)),
        out_shape=jax.ShapeDtypeStruct((N, H), jnp.float32),
    )(b_z, W1n)


def _sc_edge1_body(y1n_hbm, ei_hbm, dst2_hbm, ew_hbm, zr_hbm, zn_hbm,
                   s1a_out, s1b_out, deg_out,
                   se0, se1, se2, de0, de1, de2, we0, we1, we2,
                   r0, r1, deg_acc, s1_sh,
                   ee0, ee1, ee2, g0, g1, s0, s1sem):
    cid = lax.axis_index("c")
    sid = lax.axis_index("s")
    wid = sid * NC + cid
    ebase = wid * EPW
    src_e = [se0, se1, se2]
    dst_e = [de0, de1, de2]
    ew_e = [we0, we1, we2]
    esems = [ee0, ee1, ee2]
    rows = [r0, r1]
    gsems = [g0, g1]
    ssems = [s0, s1sem]
    z16 = jnp.zeros((16,), jnp.float32)
    one16 = jnp.ones((16,), jnp.float32)

    def stage(m):
        b = m % 3
        return [
            pltpu.async_copy(ei_hbm.at[0, pl.ds(ebase + m * C, C)],
                             src_e[b], esems[b]),
            pltpu.async_copy(dst2_hbm.at[pl.ds(wid * DPR + m * NSUB, NSUB)],
                             dst_e[b], esems[b]),
            pltpu.async_copy(ew_hbm.at[pl.ds(ebase + m * C, C)],
                             ew_e[b], esems[b]),
        ]

    def gather(m):
        b = m % 2
        return [
            pltpu.async_copy(
                y1n_hbm.at[src_e[m % 3].at[pl.ds(j * SUB, SUB)]],
                rows[b].at[pl.ds(j * SUB, SUB)], gsems[b])
            for j in range(NSUB)
        ]

    def scatter(m):
        b = m % 2
        return [
            pltpu.async_copy(rows[b].at[pl.ds(j * SUB, SUB)],
                             s1_sh.at[dst_e[m % 3].at[j]], ssems[b],
                             add=True)
            for j in range(NSUB)
        ]

    ed = {0: stage(0), 1: stage(1)}
    z1 = pltpu.async_copy(zn_hbm, deg_acc, g0)
    z2 = pltpu.async_copy(zr_hbm, s1_sh.at[pl.ds(sid * ROWS_PT, ROWS_PT)],
                          g1)
    z1.wait()
    z2.wait()
    for d in ed.pop(0):
        d.wait()
    gd = {0: gather(0)}
    plsc.subcore_barrier()

    sd = {}
    for m in range(NCHUNK):
        b = m % 2
        eb = m % 3
        if m >= 1:
            for d in sd.pop(m - 1):
                d.wait()
        if m + 2 < NCHUNK:
            ed[m + 2] = stage(m + 2)
        if m + 1 < NCHUNK:
            for d in ed.pop(m + 1):
                d.wait()
            gd[m + 1] = gather(m + 1)
        for d in gd.pop(m):
            d.wait()

        @plsc.parallel_loop(0, C, unroll=4)
        def scale(e):
            eb16 = jnp.full((16,), e, jnp.int32)
            wv = plsc.load_gather(ew_e[eb], [eb16])
            for k in range(4):
                v = rows[b][e, pl.ds(k * 16, 16)]
                rows[b][e, pl.ds(k * 16, 16)] = v * wv

        sd[m] = scatter(m)

        for jj in range(NSUB):
            def deg(q, c, _jj=jj):
                d16 = dst_e[eb][_jj, pl.ds(q * 16, 16)]
                plsc.addupdate_scatter(deg_acc, [d16], one16)
                return c

            lax.fori_loop(0, SUB // 16, deg, 0, unroll=5)

    for m in sorted(sd):
        for d in sd[m]:
            d.wait()
    plsc.subcore_barrier()

    rbase = sid * ROWS_PT
    nlast = N - 15 * ROWS_PT
    for cc, ref in ((0, s1a_out), (1, s1b_out)):
        @pl.when((cid == cc) & (rbase + ROWS_PT <= N))
        def _(ref=ref):
            pltpu.sync_copy(s1_sh.at[pl.ds(rbase, ROWS_PT)],
                            ref.at[pl.ds(rbase, ROWS_PT)])

        @pl.when((cid == cc) & (rbase + ROWS_PT > N))
        def _(ref=ref):
            pltpu.sync_copy(s1_sh.at[pl.ds(rbase, nlast)],
                            ref.at[pl.ds(rbase, nlast)])

    pltpu.sync_copy(deg_acc, deg_out.at[wid])


def _sc_edge1(y1n, ei, dst2d, ew, zr, zn):
    mesh = plsc.VectorSubcoreMesh(core_axis_name="c", subcore_axis_name="s")
    f = functools.partial(
        pl.kernel,
        compiler_params=_SC_PARAMS,
        out_type=[
            jax.ShapeDtypeStruct((N, H), jnp.float32),
            jax.ShapeDtypeStruct((N, H), jnp.float32),
            jax.ShapeDtypeStruct((NW, NPAD), jnp.float32),
        ],
        mesh=mesh,
        scratch_types=(
            [pltpu.VMEM((C,), jnp.int32)] * 3
            + [pltpu.VMEM((NSUB, SUB), jnp.int32)] * 3
            + [pltpu.VMEM((C,), jnp.float32)] * 3
            + [
                pltpu.VMEM((C, H), jnp.float32),
                pltpu.VMEM((C, H), jnp.float32),
                pltpu.VMEM((NPAD,), jnp.float32),
                pltpu.VMEM_SHARED((NPAD, H), jnp.float32),
            ]
            + [pltpu.SemaphoreType.DMA] * 7
        ),
    )(_sc_edge1_body)
    return f(y1n, ei, dst2d, ew, zr, zn)


def _tc_mid_body(bz_ref, w1s_ref, b1_ref, s1a_ref, s1b_ref, deg_ref,
                 w2n_ref, w2s_ref, b2_ref, y2_ref, t2_ref, dn_ref):
    denom = jnp.maximum(jnp.sum(deg_ref[...], axis=1, keepdims=True), 1.0)
    s1 = s1a_ref[...] + s1b_ref[...]
    a1s = (jnp.dot(bz_ref[...], w1s_ref[...],
                   preferred_element_type=jnp.float32) + b1_ref[...])
    h = jnp.tanh(a1s + s1 / denom)
    y2_ref[...] = jnp.dot(h, w2n_ref[...], preferred_element_type=jnp.float32)
    t2_ref[...] = (
        jnp.dot(h, w2s_ref[...], preferred_element_type=jnp.float32)
        + b2_ref[...]
    )
    dn_ref[...] = denom


def _tc_mid(b_z, W1s, b1, s1a, s1b, degT, W2n, W2s, b2):
    R = 1000
    return pl.pallas_call(
        _tc_mid_body,
        grid=(N // R,),
        in_specs=[
            pl.BlockSpec((R, D), lambda j: (j, 0)),
            pl.BlockSpec((D, H), lambda j: (0, 0)),
            pl.BlockSpec((H,), lambda j: (0,)),
            pl.BlockSpec((R, H), lambda j: (j, 0)),
            pl.BlockSpec((R, H), lambda j: (j, 0)),
            pl.BlockSpec((R, NW), lambda j: (j, 0)),
            pl.BlockSpec((H, 1), lambda j: (0, 0)),
            pl.BlockSpec((H, 1), lambda j: (0, 0)),
            pl.BlockSpec((1,), lambda j: (0,)),
        ],
        out_specs=[
            pl.BlockSpec((R, 1), lambda j: (j, 0)),
            pl.BlockSpec((R, 1), lambda j: (j, 0)),
            pl.BlockSpec((R, 1), lambda j: (j, 0)),
        ],
        out_shape=[
            jax.ShapeDtypeStruct((N, 1), jnp.float32),
            jax.ShapeDtypeStruct((N, 1), jnp.float32),
            jax.ShapeDtypeStruct((N, 1), jnp.float32),
        ],
    )(b_z, W1s, b1, s1a, s1b, degT, W2n, W2s, b2)


def _sc_edge2_body(y2_hbm, ei_hbm, ew_hbm, zn_hbm, s2_out,
                   y2_b, src_b, dst_b, ew_b, acc, esem):
    cid = lax.axis_index("c")
    sid = lax.axis_index("s")
    wid = sid * NC + cid
    ebase = wid * EPW

    e0 = pltpu.async_copy(y2_hbm, y2_b, esem)
    e1 = pltpu.async_copy(ei_hbm.at[0, pl.ds(ebase, EPW)], src_b, esem)
    e2 = pltpu.async_copy(ei_hbm.at[1, pl.ds(ebase, EPW)], dst_b, esem)
    e3 = pltpu.async_copy(ew_hbm.at[pl.ds(ebase, EPW)], ew_b, esem)
    e4 = pltpu.async_copy(zn_hbm, acc, esem)
    e4.wait()
    e0.wait()
    e1.wait()
    e2.wait()
    e3.wait()

    def grp(g, c):
        s16 = src_b[pl.ds(g * 16, 16)]
        d16 = dst_b[pl.ds(g * 16, 16)]
        vals = plsc.load_gather(y2_b, [s16])
        w16 = ew_b[pl.ds(g * 16, 16)]
        plsc.addupdate_scatter(acc, [d16], vals * w16)
        return c

    lax.fori_loop(0, EPW // 16, grp, 0)
    pltpu.sync_copy(acc, s2_out.at[wid])


def _sc_edge2(y2, ei, ew, zn):
    mesh = plsc.VectorSubcoreMesh(core_axis_name="c", subcore_axis_name="s")
    f = functools.partial(
        pl.kernel,
        compiler_params=_SC_PARAMS,
        out_type=[jax.ShapeDtypeStruct((NW, NPAD), jnp.float32)],
        mesh=mesh,
        scratch_types=[
            pltpu.VMEM((N,), jnp.float32),
            pltpu.VMEM((EPW,), jnp.int32),
            pltpu.VMEM((EPW,), jnp.int32),
            pltpu.VMEM((EPW,), jnp.float32),
            pltpu.VMEM((NPAD,), jnp.float32),
            pltpu.SemaphoreType.DMA,
        ],
    )(_sc_edge2_body)
    return f(y2, ei, ew, zn)[0]


def _tc_post_body(t2_ref, s2_ref, dn_ref, out_ref):
    s2 = jnp.sum(s2_ref[...], axis=1, keepdims=True)
    out_ref[...] = t2_ref[...] + s2 / dn_ref[...]


def _tc_post(t2, s2T, dn):
    return pl.pallas_call(
        _tc_post_body,
        grid=(1,),
        in_specs=[
            pl.BlockSpec((N, 1), lambda j: (0, 0)),
            pl.BlockSpec((N, NW), lambda j: (0, 0)),
            pl.BlockSpec((N, 1), lambda j: (0, 0)),
        ],
        out_specs=pl.BlockSpec((N, 1), lambda j: (0, 0)),
        out_shape=jax.ShapeDtypeStruct((N, 1), jnp.float32),
    )(t2, s2T, dn)


def kernel(b_z, edge_index, b_edge_weight, b_size, W1s, W1n, b1, W2s, W2n, b2):
    ew = b_edge_weight
    dst2d = edge_index[1].reshape(E // SUB, SUB)

    zr = jnp.zeros((ROWS_PT, H), jnp.float32)
    zn = jnp.zeros((NPAD,), jnp.float32)
    y1n = _tc_pre(b_z, W1n)
    s1a, s1b, degp = _sc_edge1(y1n, edge_index, dst2d, ew, zr, zn)
    degT = degp.T
    y2, t2, dn = _tc_mid(b_z, W1s, b1, s1a, s1b, degT, W2n, W2s, b2)
    s2p = _sc_edge2(y2.reshape(N), edge_index, ew, zn)
    out = _tc_post(t2, s2p.T, dn)
    return out.reshape(100, 100)
